# baseline (speedup 1.0000x reference)
import functools

import jax
import jax.numpy as jnp
from jax import lax
from jax.experimental import pallas as pl
from jax.experimental.pallas import tpu as pltpu
from jax.experimental.pallas import tpu_sc as plsc

B = 16384
H = 16
D = 128
CH = 128
U = 100000
RP_COLS = 8192
RP_CW = RP_COLS // 8
RP_GRID = (U + RP_COLS - 1) // RP_COLS
U_PAD = RP_GRID * RP_COLS


def _repack_body(xu_ref, xi_ref, ou_ref, oi_ref):
    for ref, o in ((xu_ref, ou_ref), (xi_ref, oi_ref)):
        x = ref[...]
        xs = jnp.concatenate(
            [x[:, c * RP_CW:(c + 1) * RP_CW] for c in range(8)], axis=0)
        o[...] = xs.T


def _repack_call(gt_u, gt_i):
    out = pl.pallas_call(
        _repack_body,
        grid=(RP_GRID,),
        in_specs=[
            pl.BlockSpec((H, RP_COLS), lambda i: (0, i)),
            pl.BlockSpec((H, RP_COLS), lambda i: (0, i)),
        ],
        out_specs=[
            pl.BlockSpec((RP_CW, 128), lambda i: (i, 0)),
            pl.BlockSpec((RP_CW, 128), lambda i: (i, 0)),
        ],
        out_shape=[
            jax.ShapeDtypeStruct((U_PAD // 8, 128), jnp.float32),
            jax.ShapeDtypeStruct((U_PAD // 8, 128), jnp.float32),
        ],
    )(gt_u, gt_i)
    return out


def _make_sc_mlp_gather(nc, ns):
    nw = nc * ns
    bpw = B // nw
    nchunk = bpw // CH
    mesh = plsc.VectorSubcoreMesh(core_axis_name="c", subcore_axis_name="s")

    @functools.partial(
        pl.kernel,
        mesh=mesh,
        compiler_params=pltpu.CompilerParams(use_tc_tiling_on_sc=False, needs_layout_passes=False),
        cost_estimate=pl.CostEstimate(
            flops=0, bytes_accessed=4 * B * D * 4, transcendentals=0),
        out_type=[
            jax.ShapeDtypeStruct((B, D), jnp.float32),
            jax.ShapeDtypeStruct((B, D), jnp.float32),
        ],
        scratch_types=[
            pltpu.VMEM((nchunk, CH), jnp.int32),
            pltpu.VMEM((nchunk, CH), jnp.int32),
            pltpu.VMEM((CH, D), jnp.float32),
            pltpu.VMEM((CH, D), jnp.float32),
            pltpu.VMEM((CH, D), jnp.float32),
            pltpu.VMEM((CH, D), jnp.float32),
            pltpu.SemaphoreType.DMA,
            pltpu.SemaphoreType.DMA,
            pltpu.SemaphoreType.DMA,
            pltpu.SemaphoreType.DMA,
        ],
    )
    def gather_k(uidx_hbm, iidx_hbm, mu_hbm, mi_hbm,
                 um_out, im_out,
                 uidx_v, iidx_v, u0, u1, i0, i1, su0, su1, si0, si1):
        wid = lax.axis_index("s") * nc + lax.axis_index("c")
        base = wid * bpw
        pltpu.sync_copy(uidx_hbm.at[wid], uidx_v)
        pltpu.sync_copy(iidx_hbm.at[wid], iidx_v)
        ubuf, ibuf = (u0, u1), (i0, i1)
        usem, isem = (su0, su1), (si0, si1)
        cps = {}
        for c in range(2):
            cps[("u", c)] = pltpu.async_copy(
                mu_hbm.at[uidx_v.at[c]], ubuf[c % 2], usem[c % 2])
            cps[("i", c)] = pltpu.async_copy(
                mi_hbm.at[iidx_v.at[c]], ibuf[c % 2], isem[c % 2])
        for c in range(nchunk):
            row = base + c * CH
            cps[("u", c)].wait()
            pltpu.sync_copy(ubuf[c % 2], um_out.at[pl.ds(row, CH)])
            if c + 2 < nchunk:
                cps[("u", c + 2)] = pltpu.async_copy(
                    mu_hbm.at[uidx_v.at[c + 2]], ubuf[c % 2], usem[c % 2])
            cps[("i", c)].wait()
            pltpu.sync_copy(ibuf[c % 2], im_out.at[pl.ds(row, CH)])
            if c + 2 < nchunk:
                cps[("i", c + 2)] = pltpu.async_copy(
                    mi_hbm.at[iidx_v.at[c + 2]], ibuf[c % 2], isem[c % 2])

    return gather_k


def _make_sc_gmf(nc, ns):
    nw = nc * ns
    bpw = B // nw
    nchunk = bpw // CH
    ngrp = bpw // 16
    mesh = plsc.VectorSubcoreMesh(core_axis_name="c", subcore_axis_name="s")

    @functools.partial(
        pl.kernel,
        mesh=mesh,
        compiler_params=pltpu.CompilerParams(use_tc_tiling_on_sc=False, needs_layout_passes=False),
        out_type=jax.ShapeDtypeStruct((B,), jnp.float32),
        scratch_types=[
            pltpu.VMEM((nchunk, CH), jnp.int32),
            pltpu.VMEM((nchunk, CH), jnp.int32),
            pltpu.VMEM((nchunk, CH), jnp.int32),
            pltpu.VMEM((nchunk, CH), jnp.int32),
            pltpu.VMEM((bpw, H), jnp.float32),
            pltpu.VMEM((bpw, H), jnp.float32),
            pltpu.VMEM((H,), jnp.float32),
            pltpu.VMEM((H,), jnp.float32),
            pltpu.VMEM((bpw,), jnp.float32),
            pltpu.SemaphoreType.DMA,
        ],
    )
    def gmf_k(uidx_hbm, iidx_hbm, gu_hbm, gi_hbm, wg_hbm, bo_hbm,
              zg_out,
              uidx_v, iidx_v, tu_v, ti_v, gu_v, gi_v, wg_v, bo_v, zg_v, sem):
        wid = lax.axis_index("s") * nc + lax.axis_index("c")
        base = wid * bpw
        pltpu.sync_copy(uidx_hbm.at[wid], uidx_v)
        pltpu.sync_copy(iidx_hbm.at[wid], iidx_v)
        pltpu.sync_copy(wg_hbm, wg_v)
        pltpu.sync_copy(bo_hbm, bo_v)
        for c in range(nchunk):
            for o in range(CH // 16):
                for src, dst in ((uidx_v, tu_v), (iidx_v, ti_v)):
                    v = src[c, pl.ds(o * 16, 16)]
                    w = ((v >> 13) * 8192 + (v & 1023) * 8 + ((v >> 10) & 7))
                    dst[c, pl.ds(o * 16, 16)] = w
        cps = []
        for c in range(nchunk):
            cps.append(pltpu.async_copy(
                gu_hbm.at[tu_v.at[c]], gu_v.at[pl.ds(c * CH, CH)], sem))
            cps.append(pltpu.async_copy(
                gi_hbm.at[ti_v.at[c]], gi_v.at[pl.ds(c * CH, CH)], sem))
        boutv = bo_v[...]
        wg_cols = [plsc.load_gather(wg_v, [jnp.full((16,), k, jnp.int32)])
                   for k in range(H)]
        iota16 = lax.iota(jnp.int32, 16)
        gpc = CH // 16

        def grp(g, _):
            ridx = g * 16 + iota16
            acc = boutv
            for k in range(H):
                cidx = jnp.full((16,), k, jnp.int32)
                ucol = plsc.load_gather(gu_v, [ridx, cidx])
                icol = plsc.load_gather(gi_v, [ridx, cidx])
                acc = acc + ucol * icol * wg_cols[k]
            zg_v[pl.ds(g * 16, 16)] = acc
            return ()

        for c in range(nchunk):
            cps[2 * c].wait()
            cps[2 * c + 1].wait()
            lax.fori_loop(c * gpc, (c + 1) * gpc, grp, (), unroll=False)
        pltpu.sync_copy(zg_v, zg_out.at[pl.ds(base, bpw)])

    return gmf_k


def _bf(x):
    return x.astype(jnp.bfloat16)


def _mlp_body(um_ref, im_ref, w0_ref, b0_ref, w1_ref, b1_ref,
              w2_ref, b2_ref, w3_ref, b3_ref, wx_ref, out_ref):
    w0 = w0_ref[...]
    h = jnp.dot(_bf(um_ref[...]), w0[:D, :],
                preferred_element_type=jnp.float32)
    h = h + jnp.dot(_bf(im_ref[...]), w0[D:, :],
                    preferred_element_type=jnp.float32)
    h = jax.nn.relu(h + b0_ref[...])
    for w_ref, b_ref in ((w1_ref, b1_ref), (w2_ref, b2_ref), (w3_ref, b3_ref)):
        h = jax.nn.relu(jnp.dot(_bf(h), w_ref[...],
                                preferred_element_type=jnp.float32) + b_ref[...])
    out_ref[...] = jnp.sum(h * wx_ref[...], axis=1)


def _head_body(zm_ref, zg_ref, out_ref):
    out_ref[...] = jax.nn.sigmoid(zm_ref[...] + zg_ref[...])


def _mlp_call(um, im, w0, b0, w1, b1, w2, b2, w3, b3, wx):
    bm = 2048
    grid = (B // bm,)

    def full_block(a):
        return pl.BlockSpec(a.shape, lambda i: (0,) * a.ndim)

    return pl.pallas_call(
        _mlp_body,
        grid=grid,
        in_specs=[
            pl.BlockSpec((bm, D), lambda i: (i, 0)),
            pl.BlockSpec((bm, D), lambda i: (i, 0)),
            full_block(w0), full_block(b0), full_block(w1), full_block(b1),
            full_block(w2), full_block(b2), full_block(w3), full_block(b3),
            full_block(wx),
        ],
        out_specs=pl.BlockSpec((bm,), lambda i: (i,)),
        out_shape=jax.ShapeDtypeStruct((B,), jnp.float32),
    )(um, im, w0, b0, w1, b1, w2, b2, w3, b3, wx)


def _head_call(zm, zg):
    return pl.pallas_call(
        _head_body,
        grid=(1,),
        in_specs=[
            pl.BlockSpec((B,), lambda i: (0,)),
            pl.BlockSpec((B,), lambda i: (0,)),
        ],
        out_specs=pl.BlockSpec((B,), lambda i: (0,)),
        out_shape=jax.ShapeDtypeStruct((B,), jnp.float32),
    )(zm, zg)


def kernel(user, item, gmf_user_emb, gmf_item_emb, mlp_user_emb, mlp_item_emb,
           W0, b0, W1, b1, W2, b2, W3, b3, Wout, bout):
    info = plsc.get_sparse_core_info()
    nc, ns = info.num_cores, info.num_subcores
    nw = nc * ns
    nchunk = B // nw // CH
    uidx = user.astype(jnp.int32).reshape(nw, nchunk, CH)
    iidx = item.astype(jnp.int32).reshape(nw, nchunk, CH)

    ru, ri = _repack_call(gmf_user_emb.T, gmf_item_emb.T)
    gu = ru.reshape(U_PAD, H)
    gi = ri.reshape(U_PAD, H)

    wg = Wout[:H, 0]
    boutv = jnp.broadcast_to(bout, (H,))
    zg = _make_sc_gmf(nc, ns)(uidx, iidx, gu, gi, wg, boutv)

    um, im = _make_sc_mlp_gather(nc, ns)(uidx, iidx, mlp_user_emb, mlp_item_emb)

    bf = jnp.bfloat16
    wx = Wout[H:, 0].reshape(1, H)
    zm = _mlp_call(um, im,
                   W0.astype(bf), b0.reshape(1, -1), W1.astype(bf),
                   b1.reshape(1, -1), W2.astype(bf), b2.reshape(1, -1),
                   W3.astype(bf), b3.reshape(1, -1), wx)
    return _head_call(zm, zg)


# speedup vs baseline: 1.0069x; 1.0069x over previous
import functools

import jax
import jax.numpy as jnp
from jax import lax
from jax.experimental import pallas as pl
from jax.experimental.pallas import tpu as pltpu
from jax.experimental.pallas import tpu_sc as plsc

B = 16384
H = 16
D = 128
CH = 128
U = 100000
RP_COLS = 8192
RP_CW = RP_COLS // 8
RP_GRID = (U + RP_COLS - 1) // RP_COLS
U_PAD = RP_GRID * RP_COLS


def _repack_body(xu_ref, xi_ref, ou_ref, oi_ref):
    for ref, o in ((xu_ref, ou_ref), (xi_ref, oi_ref)):
        x = ref[...]
        xs = jnp.concatenate(
            [x[:, c * RP_CW:(c + 1) * RP_CW] for c in range(8)], axis=0)
        o[...] = xs.T


def _repack_call(gt_u, gt_i):
    out = pl.pallas_call(
        _repack_body,
        grid=(RP_GRID,),
        in_specs=[
            pl.BlockSpec((H, RP_COLS), lambda i: (0, i)),
            pl.BlockSpec((H, RP_COLS), lambda i: (0, i)),
        ],
        out_specs=[
            pl.BlockSpec((RP_CW, 128), lambda i: (i, 0)),
            pl.BlockSpec((RP_CW, 128), lambda i: (i, 0)),
        ],
        out_shape=[
            jax.ShapeDtypeStruct((U_PAD // 8, 128), jnp.float32),
            jax.ShapeDtypeStruct((U_PAD // 8, 128), jnp.float32),
        ],
    )(gt_u, gt_i)
    return out


def _make_sc_mlp_gather(nc, ns):
    nw = nc * ns
    bpw = B // nw
    nchunk = bpw // CH
    mesh = plsc.VectorSubcoreMesh(core_axis_name="c", subcore_axis_name="s")

    @functools.partial(
        pl.kernel,
        mesh=mesh,
        compiler_params=pltpu.CompilerParams(use_tc_tiling_on_sc=False, needs_layout_passes=False),
        cost_estimate=pl.CostEstimate(
            flops=0, bytes_accessed=4 * B * D * 4, transcendentals=0),
        out_type=[
            jax.ShapeDtypeStruct((B, D), jnp.float32),
            jax.ShapeDtypeStruct((B, D), jnp.float32),
        ],
        scratch_types=[
            pltpu.VMEM((nchunk, CH), jnp.int32),
            pltpu.VMEM((nchunk, CH), jnp.int32),
            pltpu.VMEM((CH, D), jnp.float32),
            pltpu.VMEM((CH, D), jnp.float32),
            pltpu.VMEM((CH, D), jnp.float32),
            pltpu.VMEM((CH, D), jnp.float32),
            pltpu.SemaphoreType.DMA,
            pltpu.SemaphoreType.DMA,
            pltpu.SemaphoreType.DMA,
            pltpu.SemaphoreType.DMA,
        ],
    )
    def gather_k(uidx_hbm, iidx_hbm, mu_hbm, mi_hbm,
                 um_out, im_out,
                 uidx_v, iidx_v, u0, u1, i0, i1, su0, su1, si0, si1):
        wid = lax.axis_index("s") * nc + lax.axis_index("c")
        base = wid * bpw
        pltpu.sync_copy(uidx_hbm.at[wid], uidx_v)
        pltpu.sync_copy(iidx_hbm.at[wid], iidx_v)
        ubuf, ibuf = (u0, u1), (i0, i1)
        usem, isem = (su0, su1), (si0, si1)
        cps = {}
        for c in range(2):
            cps[("u", c)] = pltpu.async_copy(
                mu_hbm.at[uidx_v.at[c]], ubuf[c % 2], usem[c % 2])
            cps[("i", c)] = pltpu.async_copy(
                mi_hbm.at[iidx_v.at[c]], ibuf[c % 2], isem[c % 2])
        for c in range(nchunk):
            row = base + c * CH
            cps[("u", c)].wait()
            pltpu.sync_copy(ubuf[c % 2], um_out.at[pl.ds(row, CH)])
            if c + 2 < nchunk:
                cps[("u", c + 2)] = pltpu.async_copy(
                    mu_hbm.at[uidx_v.at[c + 2]], ubuf[c % 2], usem[c % 2])
            cps[("i", c)].wait()
            pltpu.sync_copy(ibuf[c % 2], im_out.at[pl.ds(row, CH)])
            if c + 2 < nchunk:
                cps[("i", c + 2)] = pltpu.async_copy(
                    mi_hbm.at[iidx_v.at[c + 2]], ibuf[c % 2], isem[c % 2])

    return gather_k


def _make_sc_gmf(nc, ns):
    nw = nc * ns
    bpw = B // nw
    nchunk = bpw // CH
    ngrp = bpw // 16
    mesh = plsc.VectorSubcoreMesh(core_axis_name="c", subcore_axis_name="s")

    @functools.partial(
        pl.kernel,
        mesh=mesh,
        compiler_params=pltpu.CompilerParams(use_tc_tiling_on_sc=False, needs_layout_passes=False),
        cost_estimate=pl.CostEstimate(
            flops=3 * B * H, bytes_accessed=2 * B * H * 4, transcendentals=0),
        out_type=jax.ShapeDtypeStruct((B,), jnp.float32),
        scratch_types=[
            pltpu.VMEM((nchunk, CH), jnp.int32),
            pltpu.VMEM((nchunk, CH), jnp.int32),
            pltpu.VMEM((nchunk, CH), jnp.int32),
            pltpu.VMEM((nchunk, CH), jnp.int32),
            pltpu.VMEM((bpw, H), jnp.float32),
            pltpu.VMEM((bpw, H), jnp.float32),
            pltpu.VMEM((H,), jnp.float32),
            pltpu.VMEM((H,), jnp.float32),
            pltpu.VMEM((bpw,), jnp.float32),
            pltpu.SemaphoreType.DMA,
        ],
    )
    def gmf_k(uidx_hbm, iidx_hbm, gu_hbm, gi_hbm, wg_hbm, bo_hbm,
              zg_out,
              uidx_v, iidx_v, tu_v, ti_v, gu_v, gi_v, wg_v, bo_v, zg_v, sem):
        wid = lax.axis_index("s") * nc + lax.axis_index("c")
        base = wid * bpw
        pltpu.sync_copy(uidx_hbm.at[wid], uidx_v)
        pltpu.sync_copy(iidx_hbm.at[wid], iidx_v)
        pltpu.sync_copy(wg_hbm, wg_v)
        pltpu.sync_copy(bo_hbm, bo_v)
        for c in range(nchunk):
            for o in range(CH // 16):
                for src, dst in ((uidx_v, tu_v), (iidx_v, ti_v)):
                    v = src[c, pl.ds(o * 16, 16)]
                    w = ((v >> 13) * 8192 + (v & 1023) * 8 + ((v >> 10) & 7))
                    dst[c, pl.ds(o * 16, 16)] = w
        cps = []
        for c in range(nchunk):
            cps.append(pltpu.async_copy(
                gu_hbm.at[tu_v.at[c]], gu_v.at[pl.ds(c * CH, CH)], sem))
            cps.append(pltpu.async_copy(
                gi_hbm.at[ti_v.at[c]], gi_v.at[pl.ds(c * CH, CH)], sem))
        boutv = bo_v[...]
        wg_cols = [plsc.load_gather(wg_v, [jnp.full((16,), k, jnp.int32)])
                   for k in range(H)]
        iota16 = lax.iota(jnp.int32, 16)
        gpc = CH // 16

        def grp(g, _):
            ridx = g * 16 + iota16
            acc = boutv
            for k in range(H):
                cidx = jnp.full((16,), k, jnp.int32)
                ucol = plsc.load_gather(gu_v, [ridx, cidx])
                icol = plsc.load_gather(gi_v, [ridx, cidx])
                acc = acc + ucol * icol * wg_cols[k]
            zg_v[pl.ds(g * 16, 16)] = acc
            return ()

        for c in range(nchunk):
            cps[2 * c].wait()
            cps[2 * c + 1].wait()
            lax.fori_loop(c * gpc, (c + 1) * gpc, grp, (), unroll=False)
        pltpu.sync_copy(zg_v, zg_out.at[pl.ds(base, bpw)])

    return gmf_k


def _bf(x):
    return x.astype(jnp.bfloat16)


def _mlp_body(um_ref, im_ref, w0_ref, b0_ref, w1_ref, b1_ref,
              w2_ref, b2_ref, w3_ref, b3_ref, wx_ref, out_ref):
    w0 = w0_ref[...]
    h = jnp.dot(_bf(um_ref[...]), w0[:D, :],
                preferred_element_type=jnp.float32)
    h = h + jnp.dot(_bf(im_ref[...]), w0[D:, :],
                    preferred_element_type=jnp.float32)
    h = jax.nn.relu(h + b0_ref[...])
    for w_ref, b_ref in ((w1_ref, b1_ref), (w2_ref, b2_ref), (w3_ref, b3_ref)):
        h = jax.nn.relu(jnp.dot(_bf(h), w_ref[...],
                                preferred_element_type=jnp.float32) + b_ref[...])
    out_ref[...] = jnp.sum(h * wx_ref[...], axis=1)


def _head_body(zm_ref, zg_ref, out_ref):
    out_ref[...] = jax.nn.sigmoid(zm_ref[...] + zg_ref[...])


def _mlp_call(um, im, w0, b0, w1, b1, w2, b2, w3, b3, wx):
    bm = 2048
    grid = (B // bm,)

    def full_block(a):
        return pl.BlockSpec(a.shape, lambda i: (0,) * a.ndim)

    return pl.pallas_call(
        _mlp_body,
        grid=grid,
        in_specs=[
            pl.BlockSpec((bm, D), lambda i: (i, 0)),
            pl.BlockSpec((bm, D), lambda i: (i, 0)),
            full_block(w0), full_block(b0), full_block(w1), full_block(b1),
            full_block(w2), full_block(b2), full_block(w3), full_block(b3),
            full_block(wx),
        ],
        out_specs=pl.BlockSpec((bm,), lambda i: (i,)),
        out_shape=jax.ShapeDtypeStruct((B,), jnp.float32),
    )(um, im, w0, b0, w1, b1, w2, b2, w3, b3, wx)


def _head_call(zm, zg):
    return pl.pallas_call(
        _head_body,
        grid=(1,),
        in_specs=[
            pl.BlockSpec((B,), lambda i: (0,)),
            pl.BlockSpec((B,), lambda i: (0,)),
        ],
        out_specs=pl.BlockSpec((B,), lambda i: (0,)),
        out_shape=jax.ShapeDtypeStruct((B,), jnp.float32),
    )(zm, zg)


def kernel(user, item, gmf_user_emb, gmf_item_emb, mlp_user_emb, mlp_item_emb,
           W0, b0, W1, b1, W2, b2, W3, b3, Wout, bout):
    info = plsc.get_sparse_core_info()
    nc, ns = info.num_cores, info.num_subcores
    nw = nc * ns
    nchunk = B // nw // CH
    uidx = user.astype(jnp.int32).reshape(nw, nchunk, CH)
    iidx = item.astype(jnp.int32).reshape(nw, nchunk, CH)

    ru, ri = _repack_call(gmf_user_emb.T, gmf_item_emb.T)
    gu = ru.reshape(U_PAD, H)
    gi = ri.reshape(U_PAD, H)

    wg = Wout[:H, 0]
    boutv = jnp.broadcast_to(bout, (H,))
    zg = _make_sc_gmf(nc, ns)(uidx, iidx, gu, gi, wg, boutv)

    um, im = _make_sc_mlp_gather(nc, ns)(uidx, iidx, mlp_user_emb, mlp_item_emb)

    bf = jnp.bfloat16
    wx = Wout[H:, 0].reshape(1, H)
    zm = _mlp_call(um, im,
                   W0.astype(bf), b0.reshape(1, -1), W1.astype(bf),
                   b1.reshape(1, -1), W2.astype(bf), b2.reshape(1, -1),
                   W3.astype(bf), b3.reshape(1, -1), wx)
    return _head_call(zm, zg)


# speedup vs baseline: 1.2918x; 1.2829x over previous
import functools

import jax
import jax.numpy as jnp
from jax import lax
from jax.experimental import pallas as pl
from jax.experimental.pallas import tpu as pltpu
from jax.experimental.pallas import tpu_sc as plsc

B = 16384
H = 16
D = 128
CH = 128
U = 100000
RP_COLS = 8192
RP_CW = RP_COLS // 8
RP_GRID = (U + RP_COLS - 1) // RP_COLS
U_PAD = RP_GRID * RP_COLS


def _repack_body(xu_ref, xi_ref, ou_ref, oi_ref):
    for ref, o in ((xu_ref, ou_ref), (xi_ref, oi_ref)):
        x = ref[...]
        xs = jnp.concatenate(
            [x[:, c * RP_CW:(c + 1) * RP_CW] for c in range(8)], axis=0)
        o[...] = xs.T


def _repack_call(gt_u, gt_i):
    out = pl.pallas_call(
        _repack_body,
        grid=(RP_GRID,),
        in_specs=[
            pl.BlockSpec((H, RP_COLS), lambda i: (0, i)),
            pl.BlockSpec((H, RP_COLS), lambda i: (0, i)),
        ],
        out_specs=[
            pl.BlockSpec((RP_CW, 128), lambda i: (i, 0)),
            pl.BlockSpec((RP_CW, 128), lambda i: (i, 0)),
        ],
        out_shape=[
            jax.ShapeDtypeStruct((U_PAD // 8, 128), jnp.float32),
            jax.ShapeDtypeStruct((U_PAD // 8, 128), jnp.float32),
        ],
    )(gt_u, gt_i)
    return out


def _make_sc_mlp_gather(nc, ns):
    nw = nc * ns
    bpw = B // nw
    nchunk = bpw // CH
    mesh = plsc.VectorSubcoreMesh(core_axis_name="c", subcore_axis_name="s")

    @functools.partial(
        pl.kernel,
        mesh=mesh,
        compiler_params=pltpu.CompilerParams(use_tc_tiling_on_sc=False, needs_layout_passes=False),
        cost_estimate=pl.CostEstimate(
            flops=0, bytes_accessed=4 * B * D * 4, transcendentals=0),
        out_type=[
            jax.ShapeDtypeStruct((B, D), jnp.float32),
            jax.ShapeDtypeStruct((B, D), jnp.float32),
        ],
        scratch_types=[
            pltpu.VMEM((nchunk, CH), jnp.int32),
            pltpu.VMEM((nchunk, CH), jnp.int32),
            pltpu.VMEM((CH, D), jnp.float32),
            pltpu.VMEM((CH, D), jnp.float32),
            pltpu.VMEM((CH, D), jnp.float32),
            pltpu.VMEM((CH, D), jnp.float32),
            pltpu.SemaphoreType.DMA,
            pltpu.SemaphoreType.DMA,
            pltpu.SemaphoreType.DMA,
            pltpu.SemaphoreType.DMA,
        ],
    )
    def gather_k(uidx_hbm, iidx_hbm, mu_hbm, mi_hbm,
                 um_out, im_out,
                 uidx_v, iidx_v, u0, u1, i0, i1, su0, su1, si0, si1):
        wid = lax.axis_index("s") * nc + lax.axis_index("c")
        base = wid * bpw
        pltpu.sync_copy(uidx_hbm.at[wid], uidx_v)
        pltpu.sync_copy(iidx_hbm.at[wid], iidx_v)
        ubuf, ibuf = (u0, u1), (i0, i1)
        usem, isem = (su0, su1), (si0, si1)
        cps = {}
        for c in range(2):
            cps[("u", c)] = pltpu.async_copy(
                mu_hbm.at[uidx_v.at[c]], ubuf[c % 2], usem[c % 2])
            cps[("i", c)] = pltpu.async_copy(
                mi_hbm.at[iidx_v.at[c]], ibuf[c % 2], isem[c % 2])
        for c in range(nchunk):
            row = base + c * CH
            cps[("u", c)].wait()
            pltpu.sync_copy(ubuf[c % 2], um_out.at[pl.ds(row, CH)])
            if c + 2 < nchunk:
                cps[("u", c + 2)] = pltpu.async_copy(
                    mu_hbm.at[uidx_v.at[c + 2]], ubuf[c % 2], usem[c % 2])
            cps[("i", c)].wait()
            pltpu.sync_copy(ibuf[c % 2], im_out.at[pl.ds(row, CH)])
            if c + 2 < nchunk:
                cps[("i", c + 2)] = pltpu.async_copy(
                    mi_hbm.at[iidx_v.at[c + 2]], ibuf[c % 2], isem[c % 2])

    return gather_k


def _make_sc_gmf(nc, ns):
    nw = nc * ns
    bpw = B // nw
    nchunk = bpw // CH
    ngrp = bpw // 16
    mesh = plsc.VectorSubcoreMesh(core_axis_name="c", subcore_axis_name="s")

    @functools.partial(
        pl.kernel,
        mesh=mesh,
        compiler_params=pltpu.CompilerParams(use_tc_tiling_on_sc=False, needs_layout_passes=False),
        cost_estimate=pl.CostEstimate(
            flops=3 * B * H, bytes_accessed=2 * B * H * 4, transcendentals=0),
        out_type=jax.ShapeDtypeStruct((B,), jnp.float32),
        scratch_types=[
            pltpu.VMEM((nchunk, CH), jnp.int32),
            pltpu.VMEM((nchunk, CH), jnp.int32),
            pltpu.VMEM((nchunk, CH), jnp.int32),
            pltpu.VMEM((nchunk, CH), jnp.int32),
            pltpu.VMEM((bpw, H), jnp.float32),
            pltpu.VMEM((bpw, H), jnp.float32),
            pltpu.VMEM((H,), jnp.float32),
            pltpu.VMEM((H,), jnp.float32),
            pltpu.VMEM((bpw,), jnp.float32),
            pltpu.SemaphoreType.DMA,
        ],
    )
    def gmf_k(uidx_hbm, iidx_hbm, gu_hbm, gi_hbm, wg_hbm, bo_hbm,
              zg_out,
              uidx_v, iidx_v, tu_v, ti_v, gu_v, gi_v, wg_v, bo_v, zg_v, sem):
        wid = lax.axis_index("s") * nc + lax.axis_index("c")
        base = wid * bpw
        pltpu.sync_copy(uidx_hbm.at[wid], uidx_v)
        pltpu.sync_copy(iidx_hbm.at[wid], iidx_v)
        pltpu.sync_copy(wg_hbm, wg_v)
        pltpu.sync_copy(bo_hbm, bo_v)
        for c in range(nchunk):
            for o in range(CH // 16):
                for src, dst in ((uidx_v, tu_v), (iidx_v, ti_v)):
                    v = src[c, pl.ds(o * 16, 16)]
                    w = ((v >> 13) * 8192 + (v & 1023) * 8 + ((v >> 10) & 7))
                    dst[c, pl.ds(o * 16, 16)] = w
        cps = []
        for c in range(nchunk):
            cps.append(pltpu.async_copy(
                gu_hbm.at[tu_v.at[c]], gu_v.at[pl.ds(c * CH, CH)], sem))
            cps.append(pltpu.async_copy(
                gi_hbm.at[ti_v.at[c]], gi_v.at[pl.ds(c * CH, CH)], sem))
        boutv = bo_v[...]
        wg_cols = [plsc.load_gather(wg_v, [jnp.full((16,), k, jnp.int32)])
                   for k in range(H)]
        iota16 = lax.iota(jnp.int32, 16)
        gpc = CH // 16

        def grp(g, _):
            ridx = g * 16 + iota16
            acc = boutv
            for k in range(H):
                cidx = jnp.full((16,), k, jnp.int32)
                ucol = plsc.load_gather(gu_v, [ridx, cidx])
                icol = plsc.load_gather(gi_v, [ridx, cidx])
                acc = acc + ucol * icol * wg_cols[k]
            zg_v[pl.ds(g * 16, 16)] = acc
            return ()

        for c in range(nchunk):
            cps[2 * c].wait()
            cps[2 * c + 1].wait()
            lax.fori_loop(c * gpc, (c + 1) * gpc, grp, (), unroll=False)
        pltpu.sync_copy(zg_v, zg_out.at[pl.ds(base, bpw)])

    return gmf_k


def _bf(x):
    return x.astype(jnp.bfloat16)


def _mlp_body(um_ref, im_ref, w0_ref, b0_ref, w1t_ref, b1t_ref,
              w2t_ref, b2t_ref, w3t_ref, b3t_ref, wxt_ref, out_ref):
    w0 = w0_ref[...]
    h = jnp.dot(_bf(um_ref[...]), w0[:D, :],
                preferred_element_type=jnp.float32)
    h = h + jnp.dot(_bf(im_ref[...]), w0[D:, :],
                    preferred_element_type=jnp.float32)
    h = jax.nn.relu(h + b0_ref[...])
    ht = jnp.transpose(h)
    for wt_ref, bt_ref in ((w1t_ref, b1t_ref), (w2t_ref, b2t_ref),
                           (w3t_ref, b3t_ref)):
        ht = jax.nn.relu(jnp.dot(wt_ref[...], _bf(ht),
                                 preferred_element_type=jnp.float32)
                         + bt_ref[...])
    zmt = jnp.dot(wxt_ref[...], _bf(ht),
                  preferred_element_type=jnp.float32)
    out_ref[...] = zmt[jnp.newaxis]


def _head_body(zm_ref, zg_ref, out_ref):
    out_ref[...] = jax.nn.sigmoid(zm_ref[...] + zg_ref[...])


def _mlp_call(um, im, w0, b0, w1t, b1t, w2t, b2t, w3t, b3t, wxt):
    bm = 2048
    grid = (B // bm,)

    def full_block(a):
        return pl.BlockSpec(a.shape, lambda i: (0,) * a.ndim)

    return pl.pallas_call(
        _mlp_body,
        grid=grid,
        in_specs=[
            pl.BlockSpec((bm, D), lambda i: (i, 0)),
            pl.BlockSpec((bm, D), lambda i: (i, 0)),
            full_block(w0), full_block(b0), full_block(w1t), full_block(b1t),
            full_block(w2t), full_block(b2t), full_block(w3t), full_block(b3t),
            full_block(wxt),
        ],
        out_specs=pl.BlockSpec((1, 1, bm), lambda i: (i, 0, 0)),
        out_shape=jax.ShapeDtypeStruct((B // bm, 1, bm), jnp.float32),
    )(um, im, w0, b0, w1t, b1t, w2t, b2t, w3t, b3t, wxt)


def _head_call(zm, zg):
    return pl.pallas_call(
        _head_body,
        grid=(1,),
        in_specs=[
            pl.BlockSpec((B,), lambda i: (0,)),
            pl.BlockSpec((B,), lambda i: (0,)),
        ],
        out_specs=pl.BlockSpec((B,), lambda i: (0,)),
        out_shape=jax.ShapeDtypeStruct((B,), jnp.float32),
    )(zm, zg)


def kernel(user, item, gmf_user_emb, gmf_item_emb, mlp_user_emb, mlp_item_emb,
           W0, b0, W1, b1, W2, b2, W3, b3, Wout, bout):
    info = plsc.get_sparse_core_info()
    nc, ns = info.num_cores, info.num_subcores
    nw = nc * ns
    nchunk = B // nw // CH
    uidx = user.astype(jnp.int32).reshape(nw, nchunk, CH)
    iidx = item.astype(jnp.int32).reshape(nw, nchunk, CH)

    ru, ri = _repack_call(gmf_user_emb.T, gmf_item_emb.T)
    gu = ru.reshape(U_PAD, H)
    gi = ri.reshape(U_PAD, H)

    wg = Wout[:H, 0]
    boutv = jnp.broadcast_to(bout, (H,))
    zg = _make_sc_gmf(nc, ns)(uidx, iidx, gu, gi, wg, boutv)

    um, im = _make_sc_mlp_gather(nc, ns)(uidx, iidx, mlp_user_emb, mlp_item_emb)

    bf = jnp.bfloat16
    zm = _mlp_call(um, im,
                   W0.astype(bf), b0.reshape(1, -1),
                   W1.astype(bf).T, b1.reshape(-1, 1),
                   W2.astype(bf).T, b2.reshape(-1, 1),
                   W3.astype(bf).T, b3.reshape(-1, 1),
                   Wout[H:, 0].astype(bf).reshape(1, H))
    return _head_call(zm.reshape(B), zg)


# speedup vs baseline: 1.3918x; 1.0775x over previous
import functools

import jax
import jax.numpy as jnp
from jax import lax
from jax.experimental import pallas as pl
from jax.experimental.pallas import tpu as pltpu
from jax.experimental.pallas import tpu_sc as plsc

B = 16384
H = 16
D = 128
CH = 128
U = 100000
RP_COLS = 8192
RP_CW = RP_COLS // 8
RP_GRID = (U + RP_COLS - 1) // RP_COLS
U_PAD = RP_GRID * RP_COLS


def _repack_body(xu_ref, xi_ref, ou_ref, oi_ref):
    for ref, o in ((xu_ref, ou_ref), (xi_ref, oi_ref)):
        x = ref[...]
        xs = jnp.concatenate(
            [x[:, c * RP_CW:(c + 1) * RP_CW] for c in range(8)], axis=0)
        o[...] = xs.T


def _repack_call(gt_u, gt_i):
    out = pl.pallas_call(
        _repack_body,
        grid=(RP_GRID,),
        in_specs=[
            pl.BlockSpec((H, RP_COLS), lambda i: (0, i)),
            pl.BlockSpec((H, RP_COLS), lambda i: (0, i)),
        ],
        out_specs=[
            pl.BlockSpec((RP_CW, 128), lambda i: (i, 0)),
            pl.BlockSpec((RP_CW, 128), lambda i: (i, 0)),
        ],
        out_shape=[
            jax.ShapeDtypeStruct((U_PAD // 8, 128), jnp.float32),
            jax.ShapeDtypeStruct((U_PAD // 8, 128), jnp.float32),
        ],
    )(gt_u, gt_i)
    return out


def _make_sc_mlp_gather(nc, ns):
    nw = nc * ns
    bpw = B // nw
    nchunk = bpw // CH
    mesh = plsc.VectorSubcoreMesh(core_axis_name="c", subcore_axis_name="s")

    @functools.partial(
        pl.kernel,
        mesh=mesh,
        compiler_params=pltpu.CompilerParams(use_tc_tiling_on_sc=False, needs_layout_passes=False),
        cost_estimate=pl.CostEstimate(
            flops=0, bytes_accessed=4 * B * D * 4, transcendentals=0),
        out_type=[
            jax.ShapeDtypeStruct((B, D), jnp.float32),
            jax.ShapeDtypeStruct((B, D), jnp.float32),
        ],
        scratch_types=[
            pltpu.VMEM((nchunk, CH), jnp.int32),
            pltpu.VMEM((nchunk, CH), jnp.int32),
            pltpu.VMEM((CH, D), jnp.float32),
            pltpu.VMEM((CH, D), jnp.float32),
            pltpu.VMEM((CH, D), jnp.float32),
            pltpu.VMEM((CH, D), jnp.float32),
            pltpu.SemaphoreType.DMA,
            pltpu.SemaphoreType.DMA,
            pltpu.SemaphoreType.DMA,
            pltpu.SemaphoreType.DMA,
        ],
    )
    def gather_k(uidx_hbm, iidx_hbm, mu_hbm, mi_hbm,
                 um_out, im_out,
                 uidx_v, iidx_v, u0, u1, i0, i1, su0, su1, si0, si1):
        wid = lax.axis_index("s") * nc + lax.axis_index("c")
        base = wid * bpw
        pltpu.sync_copy(uidx_hbm.at[wid], uidx_v)
        pltpu.sync_copy(iidx_hbm.at[wid], iidx_v)
        ubuf, ibuf = (u0, u1), (i0, i1)
        usem, isem = (su0, su1), (si0, si1)
        cps = {}
        for c in range(2):
            cps[("u", c)] = pltpu.async_copy(
                mu_hbm.at[uidx_v.at[c]], ubuf[c % 2], usem[c % 2])
            cps[("i", c)] = pltpu.async_copy(
                mi_hbm.at[iidx_v.at[c]], ibuf[c % 2], isem[c % 2])
        for c in range(nchunk):
            row = base + c * CH
            cps[("u", c)].wait()
            pltpu.sync_copy(ubuf[c % 2], um_out.at[pl.ds(row, CH)])
            if c + 2 < nchunk:
                cps[("u", c + 2)] = pltpu.async_copy(
                    mu_hbm.at[uidx_v.at[c + 2]], ubuf[c % 2], usem[c % 2])
            cps[("i", c)].wait()
            pltpu.sync_copy(ibuf[c % 2], im_out.at[pl.ds(row, CH)])
            if c + 2 < nchunk:
                cps[("i", c + 2)] = pltpu.async_copy(
                    mi_hbm.at[iidx_v.at[c + 2]], ibuf[c % 2], isem[c % 2])

    return gather_k


def _make_sc_gmf(nc, ns):
    nw = nc * ns
    bpw = B // nw
    nchunk = bpw // CH
    ngrp = bpw // 16
    mesh = plsc.VectorSubcoreMesh(core_axis_name="c", subcore_axis_name="s")

    @functools.partial(
        pl.kernel,
        mesh=mesh,
        compiler_params=pltpu.CompilerParams(use_tc_tiling_on_sc=False, needs_layout_passes=False),
        cost_estimate=pl.CostEstimate(
            flops=3 * B * H, bytes_accessed=2 * B * H * 4, transcendentals=0),
        out_type=jax.ShapeDtypeStruct((B,), jnp.float32),
        scratch_types=[
            pltpu.VMEM((nchunk, CH), jnp.int32),
            pltpu.VMEM((nchunk, CH), jnp.int32),
            pltpu.VMEM((nchunk, CH), jnp.int32),
            pltpu.VMEM((nchunk, CH), jnp.int32),
            pltpu.VMEM((bpw, H), jnp.float32),
            pltpu.VMEM((bpw, H), jnp.float32),
            pltpu.VMEM((1, 2 * H), jnp.float32),
            pltpu.VMEM((bpw,), jnp.float32),
            pltpu.SemaphoreType.DMA,
        ],
    )
    def gmf_k(uidx_hbm, iidx_hbm, gu_hbm, gi_hbm, wt_hbm,
              zg_out,
              uidx_v, iidx_v, tu_v, ti_v, gu_v, gi_v, wt_v, zg_v, sem):
        wid = lax.axis_index("s") * nc + lax.axis_index("c")
        base = wid * bpw
        pltpu.sync_copy(uidx_hbm.at[wid], uidx_v)
        pltpu.sync_copy(iidx_hbm.at[wid], iidx_v)
        pltpu.sync_copy(wt_hbm, wt_v)
        for c in range(nchunk):
            for o in range(CH // 16):
                for src, dst in ((uidx_v, tu_v), (iidx_v, ti_v)):
                    v = src[c, pl.ds(o * 16, 16)]
                    w = ((v >> 13) * 8192 + (v & 1023) * 8 + ((v >> 10) & 7))
                    dst[c, pl.ds(o * 16, 16)] = w
        cps = []
        for c in range(nchunk):
            cps.append(pltpu.async_copy(
                gu_hbm.at[tu_v.at[c]], gu_v.at[pl.ds(c * CH, CH)], sem))
            cps.append(pltpu.async_copy(
                gi_hbm.at[ti_v.at[c]], gi_v.at[pl.ds(c * CH, CH)], sem))
        zero16 = jnp.full((16,), 0, jnp.int32)
        wg_cols = [plsc.load_gather(wt_v, [zero16, jnp.full((16,), k, jnp.int32)])
                   for k in range(H)]
        iota16 = lax.iota(jnp.int32, 16)
        gpc = CH // 16

        def grp(g, _):
            ridx = g * 16 + iota16
            acc = jnp.zeros((16,), jnp.float32)
            for k in range(H):
                cidx = jnp.full((16,), k, jnp.int32)
                ucol = plsc.load_gather(gu_v, [ridx, cidx])
                icol = plsc.load_gather(gi_v, [ridx, cidx])
                acc = acc + ucol * icol * wg_cols[k]
            zg_v[pl.ds(g * 16, 16)] = acc
            return ()

        for c in range(nchunk):
            cps[2 * c].wait()
            cps[2 * c + 1].wait()
            lax.fori_loop(c * gpc, (c + 1) * gpc, grp, (), unroll=False)
        pltpu.sync_copy(zg_v, zg_out.at[pl.ds(base, bpw)])

    return gmf_k


def _bf(x):
    return x.astype(jnp.bfloat16)


def _mlp_body(um_ref, im_ref, w0_ref, b0_ref, w1t_ref, b1t_ref,
              w2t_ref, b2t_ref, w3t_ref, b3t_ref, wxt_ref, out_ref):
    w0 = _bf(w0_ref[...])
    h = jnp.dot(_bf(um_ref[...]), w0[:D, :],
                preferred_element_type=jnp.float32)
    h = h + jnp.dot(_bf(im_ref[...]), w0[D:, :],
                    preferred_element_type=jnp.float32)
    h = jax.nn.relu(h + b0_ref[...])
    ht = jnp.transpose(h)
    for wt_ref, bt_ref in ((w1t_ref, b1t_ref), (w2t_ref, b2t_ref),
                           (w3t_ref, b3t_ref)):
        ht = jax.nn.relu(jnp.dot(_bf(wt_ref[...]), _bf(ht),
                                 preferred_element_type=jnp.float32)
                         + bt_ref[...])
    zmt = jnp.dot(_bf(wxt_ref[...][:, H:]), _bf(ht),
                  preferred_element_type=jnp.float32)
    out_ref[...] = zmt[jnp.newaxis]


def _head_body(zm_ref, zg_ref, bo_ref, out_ref):
    out_ref[...] = jax.nn.sigmoid(zm_ref[...] + zg_ref[...] + bo_ref[0])


def _mlp_call(um, im, w0, b0, w1t, b1t, w2t, b2t, w3t, b3t, wxt):
    bm = 2048
    grid = (B // bm,)

    def full_block(a):
        return pl.BlockSpec(a.shape, lambda i: (0,) * a.ndim)

    return pl.pallas_call(
        _mlp_body,
        grid=grid,
        in_specs=[
            pl.BlockSpec((bm, D), lambda i: (i, 0)),
            pl.BlockSpec((bm, D), lambda i: (i, 0)),
            full_block(w0), full_block(b0), full_block(w1t), full_block(b1t),
            full_block(w2t), full_block(b2t), full_block(w3t), full_block(b3t),
            full_block(wxt),
        ],
        out_specs=pl.BlockSpec((1, 1, bm), lambda i: (i, 0, 0)),
        out_shape=jax.ShapeDtypeStruct((B // bm, 1, bm), jnp.float32),
    )(um, im, w0, b0, w1t, b1t, w2t, b2t, w3t, b3t, wxt)


def _head_call(zm, zg, bout):
    return pl.pallas_call(
        _head_body,
        grid=(1,),
        in_specs=[
            pl.BlockSpec((B,), lambda i: (0,)),
            pl.BlockSpec((B,), lambda i: (0,)),
            pl.BlockSpec(memory_space=pltpu.SMEM),
        ],
        out_specs=pl.BlockSpec((B,), lambda i: (0,)),
        out_shape=jax.ShapeDtypeStruct((B,), jnp.float32),
    )(zm, zg, bout)


def kernel(user, item, gmf_user_emb, gmf_item_emb, mlp_user_emb, mlp_item_emb,
           W0, b0, W1, b1, W2, b2, W3, b3, Wout, bout):
    info = plsc.get_sparse_core_info()
    nc, ns = info.num_cores, info.num_subcores
    nw = nc * ns
    nchunk = B // nw // CH
    uidx = user.astype(jnp.int32).reshape(nw, nchunk, CH)
    iidx = item.astype(jnp.int32).reshape(nw, nchunk, CH)

    ru, ri = _repack_call(gmf_user_emb.T, gmf_item_emb.T)
    gu = ru.reshape(U_PAD, H)
    gi = ri.reshape(U_PAD, H)

    zg = _make_sc_gmf(nc, ns)(uidx, iidx, gu, gi, Wout.T)

    um, im = _make_sc_mlp_gather(nc, ns)(uidx, iidx, mlp_user_emb, mlp_item_emb)

    zm = _mlp_call(um, im,
                   W0, b0.reshape(1, -1),
                   W1.T, b1.reshape(-1, 1),
                   W2.T, b2.reshape(-1, 1),
                   W3.T, b3.reshape(-1, 1),
                   Wout.T)
    return _head_call(zm.reshape(B), zg, bout)
